# Initial kernel scaffold; baseline (speedup 1.0000x reference)
import jax
import jax.numpy as jnp
from jax.experimental import pallas as pl


def kernel(x, edge_index, W1, a1_src, a1_dst, W2, a2_src, a2_dst):
    raise NotImplementedError("write your pallas kernel here")


# speedup vs baseline: 72.3485x; 72.3485x over previous
import functools

import jax
import jax.numpy as jnp
from jax import lax
from jax.experimental import pallas as pl
from jax.experimental.pallas import tpu as pltpu
from jax.experimental.pallas import tpu_sc as plsc

ALPHA = 0.2
NC, NS, L = 2, 16, 16
NW = NC * NS
K = 80
H, D = 4, 32
ROW = H * D


def _project_body(x_ref, wf_ref, wsd_ref, t_ref, esed_ref, b_ref):
    xv = x_ref[...]
    t_ref[...] = jnp.dot(xv, wf_ref[...], preferred_element_type=jnp.float32)
    esed = jnp.dot(xv, wsd_ref[...], preferred_element_type=jnp.float32)
    esed_ref[...] = esed
    mx = jnp.max(esed, axis=0, keepdims=True)
    b4 = mx[:, 0:4] + mx[:, 4:8]
    b4 = jnp.where(b4 >= 0.0, b4, ALPHA * b4)
    b_ref[...] = jnp.concatenate(
        [b4, jnp.zeros((1, 12), jnp.float32)], axis=1)


def _project(x, wflat, wsd):
    n = x.shape[0]
    return pl.pallas_call(
        _project_body,
        out_shape=[
            jax.ShapeDtypeStruct((n, ROW), jnp.float32),
            jax.ShapeDtypeStruct((n, L), jnp.float32),
            jax.ShapeDtypeStruct((1, L), jnp.float32),
        ],
    )(x, wflat, wsd)


def _combine_body(am_ref, ad_ref, r_ref, h_ref):
    m = am_ref[0] + am_ref[1]
    d = ad_ref[0] + ad_ref[1]
    db = jnp.dot(d, r_ref[...], preferred_element_type=jnp.float32)
    h_ref[...] = jnp.maximum(m / (db + 1e-16), 0.0)


def _combine(am, ad, r16):
    n = am.shape[1]
    return pl.pallas_call(
        _combine_body,
        out_shape=jax.ShapeDtypeStruct((n, ROW), jnp.float32),
    )(am, ad, r16)


def _edge_body(n, e, src_hbm, dst_hbm, t_hbm, esed_hbm, b_hbm, z128_hbm,
               z16_hbm, am_hbm, ad_hbm, esb_v, edb_v, b_v, src_v, dst_v,
               rows_v, exb_v, accm_s, accd_s, gsem0, gsem1, ssem0, ssem1):
    cid = lax.axis_index("c")
    sid = lax.axis_index("s")
    w = cid * NS + sid
    ew = e // NW
    c = ew // K
    nrs = n // NS
    ebase = w * ew
    gsems = (gsem0, gsem1)
    ssems = (ssem0, ssem1)
    iota = lax.iota(jnp.int32, L)

    pltpu.sync_copy(b_hbm, b_v)
    pltpu.sync_copy(z128_hbm.at[pl.ds(sid * nrs, nrs)],
                    accm_s.at[pl.ds(sid * nrs, nrs)])
    pltpu.sync_copy(z16_hbm.at[pl.ds(sid * nrs, nrs)],
                    accd_s.at[pl.ds(sid * nrs, nrs)])
    def _zero_exb(k_, _):
        exb_v[0, k_, :] = jnp.zeros((L,), jnp.float32)
        exb_v[1, k_, :] = jnp.zeros((L,), jnp.float32)
        return 0
    lax.fori_loop(0, K, _zero_exb, 0)

    bvec = b_v[0, :]
    bs = [bvec[h] for h in range(H)]

    plsc.subcore_barrier()

    def load_idx(i, buf):
        off = pl.multiple_of(ebase + i * K, 8)
        pltpu.sync_copy(src_hbm.at[pl.ds(off, K)], src_v.at[buf])
        pltpu.sync_copy(dst_hbm.at[pl.ds(off, K)], dst_v.at[buf])

    def gather_copies(buf):
        return (
            pltpu.make_async_copy(
                t_hbm.at[src_v.at[buf]], rows_v.at[buf], gsems[buf]),
            pltpu.make_async_copy(
                esed_hbm.at[src_v.at[buf]], esb_v.at[buf], gsems[buf]),
            pltpu.make_async_copy(
                esed_hbm.at[dst_v.at[buf]], edb_v.at[buf], gsems[buf]),
        )

    def start_gather(buf):
        for cp in gather_copies(buf):
            cp.start()

    def wait_gather(buf):
        for cp in gather_copies(buf):
            cp.wait()

    def start_scatter(buf):
        pltpu.make_async_copy(
            rows_v.at[buf], accm_s.at[dst_v.at[buf]], ssems[buf]
        ).start(add=True)
        pltpu.make_async_copy(
            exb_v.at[buf], accd_s.at[dst_v.at[buf]], ssems[buf]
        ).start(add=True)

    def wait_scatter(buf):
        pltpu.make_async_copy(
            rows_v.at[buf], accm_s.at[dst_v.at[buf]], ssems[buf]).wait()
        pltpu.make_async_copy(
            exb_v.at[buf], accd_s.at[dst_v.at[buf]], ssems[buf]).wait()

    def compute(buf):
        def group(g, _):
            kv = g * L + iota
            for h in range(H):
                hh = jnp.full((L,), h, jnp.int32)
                es = plsc.load_gather(esb_v.at[buf], [kv, hh])
                ed = plsc.load_gather(edb_v.at[buf], [kv, hh + 4])
                z = es + ed
                z = jnp.where(z >= 0.0, z, ALPHA * z)
                ex = jnp.exp(z - bs[h])
                plsc.store_scatter(exb_v.at[buf], [kv, hh], ex)
            return 0
        lax.fori_loop(0, K // L, group, 0)

        def edge(k, _):
            exrow = exb_v[buf, k, :]
            for h in range(H):
                exf = exrow[h]
                for jj in range(2):
                    sl = pl.ds(h * D + jj * L, L)
                    rows_v[buf, k, sl] = rows_v[buf, k, sl] * exf
            return 0
        lax.fori_loop(0, K, edge, 0)

    def step(i, buf):
        nbuf = 1 - buf

        @pl.when(i + 1 < c)
        def _():
            @pl.when(i >= 1)
            def _():
                wait_scatter(nbuf)
            load_idx(i + 1, nbuf)
            start_gather(nbuf)

        wait_gather(buf)
        compute(buf)
        start_scatter(buf)

    load_idx(0, 0)
    start_gather(0)

    def pair(t, _):
        step(2 * t, 0)
        step(2 * t + 1, 1)
        return 0
    lax.fori_loop(0, c // 2, pair, 0)
    if c % 2:
        step(c - 1, (c - 1) % 2)

    wait_scatter((c - 2) % 2)
    wait_scatter((c - 1) % 2)
    plsc.subcore_barrier()

    pltpu.sync_copy(accm_s.at[pl.ds(sid * nrs, nrs)],
                    am_hbm.at[cid, pl.ds(sid * nrs, nrs)])
    pltpu.sync_copy(accd_s.at[pl.ds(sid * nrs, nrs)],
                    ad_hbm.at[cid, pl.ds(sid * nrs, nrs)])


def _edges(src, dst, t, esed, b, z128, z16):
    n = t.shape[0]
    e = src.shape[0]
    mesh = plsc.VectorSubcoreMesh(core_axis_name="c", subcore_axis_name="s")
    return pl.kernel(
        functools.partial(_edge_body, n, e),
        out_type=[
            jax.ShapeDtypeStruct((NC, n, ROW), jnp.float32),
            jax.ShapeDtypeStruct((NC, n, L), jnp.float32),
        ],
        mesh=mesh,
        compiler_params=pltpu.CompilerParams(use_tc_tiling_on_sc=False,
                                             needs_layout_passes=False),
        scratch_types=[
            pltpu.VMEM((2, K, L), jnp.float32),
            pltpu.VMEM((2, K, L), jnp.float32),
            pltpu.VMEM((1, L), jnp.float32),
            pltpu.VMEM((2, K), jnp.int32),
            pltpu.VMEM((2, K), jnp.int32),
            pltpu.VMEM((2, K, ROW), jnp.float32),
            pltpu.VMEM((2, K, L), jnp.float32),
            pltpu.VMEM_SHARED((n, ROW), jnp.float32),
            pltpu.VMEM_SHARED((n, L), jnp.float32),
            pltpu.SemaphoreType.DMA,
            pltpu.SemaphoreType.DMA,
            pltpu.SemaphoreType.DMA,
            pltpu.SemaphoreType.DMA,
        ],
    )(src, dst, t, esed, b, z128, z16)


def _prep(wmat, a_src, a_dst):
    f = wmat.shape[1]
    wflat = wmat.transpose(1, 0, 2).reshape(f, H * D)
    ws = jnp.einsum("hfd,hd->fh", wmat, a_src)
    wd = jnp.einsum("hfd,hd->fh", wmat, a_dst)
    pad = jnp.zeros((f, 8), jnp.float32)
    return wflat, jnp.concatenate([ws, wd, pad], axis=1)


def kernel(x, edge_index, W1, a1_src, a1_dst, W2, a2_src, a2_dst):
    n = x.shape[0]
    src = edge_index[0]
    dst = edge_index[1]
    w1flat, wsd1 = _prep(W1, a1_src, a1_dst)
    w2flat, wsd2 = _prep(W2, a2_src, a2_dst)
    r16 = jnp.zeros((L, ROW), jnp.float32)
    r16 = r16.at[jnp.arange(ROW) // D, jnp.arange(ROW)].set(
        1.0, indices_are_sorted=False)
    z128 = jnp.zeros((n, ROW), jnp.float32)
    z16 = jnp.zeros((n, L), jnp.float32)

    t1, esed1, b1 = _project(x, w1flat, wsd1)
    am1, ad1 = _edges(src, dst, t1, esed1, b1, z128, z16)
    h1 = _combine(am1, ad1, r16)
    t2, esed2, b2 = _project(h1, w2flat, wsd2)
    am2, ad2 = _edges(src, dst, t2, esed2, b2, z128, z16)
    return _combine(am2, ad2, r16)
